# baseline (speedup 1.0000x reference)
import functools

import jax
import jax.numpy as jnp
from jax import lax
from jax.experimental import pallas as pl
from jax.experimental.pallas import tpu as pltpu
from jax.experimental.pallas import tpu_sc as plsc

NC, NS = 2, 16
NW = NC * NS
B, F = 4096, 26
BAGS = B * F
L = 20
H = 64
BAGS_PER_W = BAGS // NW
CHUNK_BAGS = 16
N_CHUNK = BAGS_PER_W // CHUNK_BAGS
IDX_PER_CHUNK = CHUNK_BAGS * L
NBUF = 4
SUPER = 16
N_SUPER = N_CHUNK // SUPER
IDX_PER_SUPER = SUPER * IDX_PER_CHUNK
HGRP = H // 16

_mesh = plsc.VectorSubcoreMesh(core_axis_name="c", subcore_axis_name="s")


@functools.partial(
    pl.kernel,
    out_type=jax.ShapeDtypeStruct((BAGS, H), jnp.float32),
    mesh=_mesh,
    compiler_params=pltpu.CompilerParams(use_tc_tiling_on_sc=False),
    scratch_types=[
        pltpu.VMEM((2, IDX_PER_SUPER), jnp.int32),
        pltpu.VMEM((NBUF, IDX_PER_CHUNK, H), jnp.float32),
        pltpu.VMEM((NBUF, CHUNK_BAGS, H), jnp.float32),
        pltpu.SemaphoreType.DMA,
        pltpu.SemaphoreType.DMA,
        pltpu.SemaphoreType.DMA,
        pltpu.SemaphoreType.DMA,
        pltpu.SemaphoreType.DMA,
        pltpu.SemaphoreType.DMA,
        pltpu.SemaphoreType.DMA,
        pltpu.SemaphoreType.DMA,
        pltpu.SemaphoreType.DMA,
    ],
)
def _embed_kernel(x_hbm, w_hbm, out_hbm, idx_v, rows_v, acc_v, isem,
                  gsem0, gsem1, gsem2, gsem3, osem0, osem1, osem2, osem3):
    gsems = (gsem0, gsem1, gsem2, gsem3)
    osems = (osem0, osem1, osem2, osem3)
    wid = lax.axis_index("s") * NC + lax.axis_index("c")
    ibase = wid * BAGS_PER_W * L

    def fire_super(s, sbuf):
        pltpu.async_copy(
            x_hbm.at[pl.ds(ibase + s * IDX_PER_SUPER, IDX_PER_SUPER)],
            idx_v.at[sbuf],
            isem,
        )

    def wait_super(sbuf):
        pltpu.make_async_copy(
            x_hbm.at[pl.ds(0, IDX_PER_SUPER)], idx_v.at[sbuf], isem
        ).wait()

    def fire_gather(g, rbuf):
        sbuf = (g // SUPER) & 1
        off = (g % SUPER) * IDX_PER_CHUNK
        pltpu.async_copy(
            w_hbm.at[idx_v.at[sbuf, pl.ds(off, IDX_PER_CHUNK)]],
            rows_v.at[rbuf],
            gsems[rbuf],
        )

    fire_super(0, 0)
    wait_super(0)
    for rbuf in range(NBUF):
        fire_gather(rbuf, rbuf)

    def quad_body(q, carry):
        for buf in range(NBUF):
            g = NBUF * q + buf
            pltpu.make_async_copy(
                w_hbm.at[pl.ds(0, IDX_PER_CHUNK)], rows_v.at[buf], gsems[buf]
            ).wait()

            if buf == 0:
                @pl.when(jnp.logical_and((q & 3) == 0, q >> 2 < N_SUPER - 1))
                def _():
                    fire_super((q >> 2) + 1, ((q >> 2) + 1) & 1)

            @pl.when(q > 0)
            def _():
                pltpu.make_async_copy(
                    acc_v.at[buf], out_hbm.at[pl.ds(0, CHUNK_BAGS)], osems[buf]
                ).wait()

            def bag_body(i, c):
                for u in range(2):
                    bag = 2 * i + u
                    base = bag * L
                    for j in range(HGRP):
                        col = pl.ds(j * 16, 16)
                        v = [rows_v[buf, base + l, col] for l in range(L)]
                        while len(v) > 1:
                            nxt = [v[k] + v[k + 1]
                                   for k in range(0, len(v) - 1, 2)]
                            if len(v) % 2:
                                nxt.append(v[-1])
                            v = nxt
                        acc_v[buf, bag, col] = v[0] * (1.0 / L)
                return c

            lax.fori_loop(0, CHUNK_BAGS // 2, bag_body, 0)

            obase = wid * BAGS_PER_W + g * CHUNK_BAGS
            pltpu.async_copy(
                acc_v.at[buf], out_hbm.at[pl.ds(obase, CHUNK_BAGS)], osems[buf]
            )

            @pl.when(q < N_CHUNK // NBUF - 1)
            def _():
                if buf == 0:
                    @pl.when((g + NBUF) % (SUPER * IDX_PER_CHUNK // IDX_PER_CHUNK) == 0)
                    def _():
                        wait_super(((g + NBUF) // SUPER) & 1)

                fire_gather(g + NBUF, buf)
        return carry

    lax.fori_loop(0, N_CHUNK // NBUF, quad_body, 0)

    for buf in range(NBUF):
        pltpu.make_async_copy(
            acc_v.at[buf], out_hbm.at[pl.ds(0, CHUNK_BAGS)], osems[buf]
        ).wait()


def kernel(x, weight):
    xi = x.reshape(-1).astype(jnp.int32)
    out = _embed_kernel(xi, weight)
    return out.reshape(B, F, H)
